# baseline (speedup 1.0000x reference)
import functools

import jax
import jax.numpy as jnp
from jax import lax
from jax.experimental import pallas as pl
from jax.experimental.pallas import tpu as pltpu
from jax.experimental.pallas import tpu_sc as plsc

NC = 2
NS = 16
LANES = 16
TILES = NC * NS
BK = 512
CGB = 32
CAP = 12288


def _cdiv(a, b):
    return (a + b - 1) // b


def _proj_body(x_ref, wl_ref, bl_ref, wr_ref, br_ref, xl_ref, xr_ref):
    xb = x_ref[...]
    h = pl.program_id(0)
    blh = bl_ref[pl.ds(h, 1), :]
    brh = br_ref[pl.ds(h, 1), :]
    xl_ref[0] = jnp.dot(xb, wl_ref[...],
                        preferred_element_type=jnp.float32) + blh
    xr_ref[0] = jnp.dot(xb, wr_ref[...],
                        preferred_element_type=jnp.float32) + brh


def _node_proj(x_pad, Wl, bl2, Wr, br2, H, C, N1):
    D = x_pad.shape[1]
    T = 512
    bl2_shape = bl2.shape
    grid = (H, N1 // T)
    return pl.pallas_call(
        _proj_body,
        grid=grid,
        in_specs=[
            pl.BlockSpec((T, D), lambda h, i: (i, 0)),
            pl.BlockSpec((D, C), lambda h, i: (0, h)),
            pl.BlockSpec(bl2_shape, lambda h, i: (0, 0)),
            pl.BlockSpec((D, C), lambda h, i: (0, h)),
            pl.BlockSpec(bl2_shape, lambda h, i: (0, 0)),
        ],
        out_specs=[
            pl.BlockSpec((1, T, C), lambda h, i: (h, i, 0)),
            pl.BlockSpec((1, T, C), lambda h, i: (h, i, 0)),
        ],
        out_shape=[
            jax.ShapeDtypeStruct((H, N1, C), jnp.float32),
            jax.ShapeDtypeStruct((H, N1, C), jnp.float32),
        ],
    )(x_pad, Wl, bl2, Wr, br2)


def _bucket(dstp_hbm, dbuf, listb, lo, hi, ep_pad):
    iota = lax.iota(jnp.int32, LANES)
    zero16i = jnp.zeros((LANES,), jnp.int32)

    @pl.loop(0, CAP // LANES, unroll=8)
    def _(k):
        listb[pl.ds(k * LANES, LANES)] = zero16i

    @pl.loop(0, ep_pad // BK, init_carry=jnp.int32(0))
    def pos(k, pos):
        pltpu.sync_copy(dstp_hbm.at[pl.ds(k * BK, BK)], dbuf)
        for v in range(BK // LANES):
            dv = dbuf[pl.ds(v * LANES, LANES)]
            ids = iota + (k * BK + v * LANES)
            m = (dv >= lo) & (dv < hi)
            ps = jnp.minimum(pos, CAP - LANES)
            plsc.store_compressed(listb.at[pl.ds(ps, LANES)], ids, mask=m)
            pc = plsc.all_reduce_population_count(m)
            pos = pos + pc[0]
        return pos

    return pos


def _ea_stats_body(dstp_hbm, ea_hbm, ea_out, deg_out,
                   listb, dbuf, cidxb, eabuf, dstb, accea, accdeg,
                   sem1, sem2, E, ep_pad, rpt):
    cid = lax.axis_index("c")
    sid = lax.axis_index("s")
    wid = sid * NC + cid
    lo = wid * rpt
    iota = lax.iota(jnp.int32, LANES)
    zero16f = jnp.zeros((LANES,), jnp.float32)
    zero16i = jnp.zeros((LANES,), jnp.int32)
    ones16f = jnp.full((LANES,), 1.0, jnp.float32)

    @pl.loop(0, rpt, unroll=4)
    def _(r):
        for c in range(8):
            accea[r, pl.ds(c * LANES, LANES)] = zero16f
        accdeg[r, :] = zero16f

    pos = _bucket(dstp_hbm, dbuf, listb, lo, lo + rpt, ep_pad)

    @pl.loop(0, CAP // CGB)
    def _(c):
        base = c * CGB

        @pl.when(base < pos)
        def _():
            for v in range(CGB // LANES):
                idv = listb[pl.ds(base + v * LANES, LANES)]
                cidxb[pl.ds(v * LANES, LANES)] = jnp.minimum(idv, E - 1)
            c1 = pltpu.async_copy(ea_hbm.at[cidxb], eabuf, sem1)
            c2 = pltpu.async_copy(dstp_hbm.at[cidxb], dstb, sem2)
            c1.wait()
            c2.wait()
            for v in range(CGB // LANES):
                idv = listb[pl.ds(base + v * LANES, LANES)]
                inb = (iota + (base + v * LANES)) < pos
                mk = inb & (idv < E)
                dstv = dstb[pl.ds(v * LANES, LANES)]
                ridx = jnp.clip(dstv - lo, 0, rpt - 1)
                plsc.addupdate_scatter(accdeg, [ridx, zero16i], ones16f, mask=mk)
                erows = iota + v * LANES

                @pl.loop(0, 128, unroll=16)
                def _(ch):
                    chs = jnp.broadcast_to(ch, (LANES,))
                    vals = plsc.load_gather(eabuf, [erows, chs])
                    plsc.addupdate_scatter(accea, [ridx, chs], vals, mask=mk)

    pltpu.sync_copy(accea, ea_out.at[pl.ds(lo, rpt)])
    pltpu.sync_copy(accdeg, deg_out.at[pl.ds(lo, rpt)])


def _ea_stats(dstp, edge_attr, N1, EP_pad):
    E, D = edge_attr.shape
    rpt = N1 // TILES
    mesh = plsc.VectorSubcoreMesh(core_axis_name="c", subcore_axis_name="s",
                                  num_cores=NC, num_subcores=NS)
    body = functools.partial(_ea_stats_body, E=E, ep_pad=EP_pad, rpt=rpt)
    fn = pl.kernel(
        body,
        out_type=[
            jax.ShapeDtypeStruct((N1, D), jnp.float32),
            jax.ShapeDtypeStruct((N1, 16), jnp.float32),
        ],
        mesh=mesh,
        compiler_params=pltpu.CompilerParams(needs_layout_passes=False),
        scratch_types=[
            pltpu.VMEM((CAP,), jnp.int32),
            pltpu.VMEM((BK,), jnp.int32),
            pltpu.VMEM((CGB,), jnp.int32),
            pltpu.VMEM((CGB, D), jnp.float32),
            pltpu.VMEM((CGB,), jnp.int32),
            pltpu.VMEM((N1 // TILES, D), jnp.float32),
            pltpu.VMEM((N1 // TILES, 16), jnp.float32),
            pltpu.SemaphoreType.DMA,
            pltpu.SemaphoreType.DMA,
        ],
    )
    return fn(dstp, edge_attr)


def _ee_edge_body(ea_ref, we_ref, out_ref):
    out_ref[0] = jnp.dot(ea_ref[...], we_ref[...],
                         preferred_element_type=jnp.float32)


def _ee_edges(edge_attr, We, H, C):
    E, D = edge_attr.shape
    T = 512
    grid = (H, E // T)
    return pl.pallas_call(
        _ee_edge_body,
        grid=grid,
        in_specs=[
            pl.BlockSpec((T, D), lambda h, i: (i, 0)),
            pl.BlockSpec((D, C), lambda h, i: (0, h)),
        ],
        out_specs=pl.BlockSpec((1, T, C), lambda h, i: (h, i, 0)),
        out_shape=jax.ShapeDtypeStruct((H, E, C), jnp.float32),
    )(edge_attr, We)


def _ee_loop_body(ea_ref, dg_ref, we_ref, out_ref):
    deg = jnp.maximum(dg_ref[:, 0:1], 1.0)
    mean = ea_ref[...] / deg
    out_ref[0] = jnp.dot(mean, we_ref[...],
                         preferred_element_type=jnp.float32)


def _ee_loops(ea_sum, deg, We, H, C):
    N1, D = ea_sum.shape
    T = 512
    grid = (H, N1 // T)
    return pl.pallas_call(
        _ee_loop_body,
        grid=grid,
        in_specs=[
            pl.BlockSpec((T, D), lambda h, i: (i, 0)),
            pl.BlockSpec((T, 16), lambda h, i: (i, 0)),
            pl.BlockSpec((D, C), lambda h, i: (0, h)),
        ],
        out_specs=pl.BlockSpec((1, T, C), lambda h, i: (h, i, 0)),
        out_shape=jax.ShapeDtypeStruct((H, N1, C), jnp.float32),
    )(ea_sum, deg, We)


def _gat_body(dstp_hbm, gsrc_hbm, gdst_hbm, xl_hbm, xr_hbm, ee_hbm, att_hbm,
              uout, exout,
              listb, dbuf, idxeb, cidxb, sidxb, didxb, dstb,
              xlg, xrg, eeg, exb, attbuf, acc, den,
              sem1, sem2, sem3, sem4,
              H, C, ep_pad, rpt):
    cid = lax.axis_index("c")
    sid = lax.axis_index("s")
    wid = sid * NC + cid
    lo = wid * rpt
    nreg = C // LANES
    iota = lax.iota(jnp.int32, LANES)
    zero16f = jnp.zeros((LANES,), jnp.float32)
    zero16i = jnp.zeros((LANES,), jnp.int32)

    pltpu.sync_copy(att_hbm, attbuf)
    pos = _bucket(dstp_hbm, dbuf, listb, lo, lo + rpt, ep_pad)

    for h in range(H):
        atts = [attbuf[h, pl.ds(r * LANES, LANES)] for r in range(nreg)]

        @pl.loop(0, rpt, unroll=4)
        def _(r):
            for c in range(nreg):
                acc[r, pl.ds(c * LANES, LANES)] = zero16f
            den[r, :] = zero16f

        @pl.loop(0, CAP // CGB)
        def _(c):
            base = c * CGB

            @pl.when(base < pos)
            def _():
                for v in range(CGB // LANES):
                    idv = listb[pl.ds(base + v * LANES, LANES)]
                    idxeb[pl.ds(v * LANES, LANES)] = idv + h * ep_pad
                    cidxb[pl.ds(v * LANES, LANES)] = idv
                c1 = pltpu.async_copy(gsrc_hbm.at[idxeb], sidxb, sem1)
                c2 = pltpu.async_copy(gdst_hbm.at[idxeb], didxb, sem2)
                c3 = pltpu.async_copy(ee_hbm.at[idxeb], eeg, sem3)
                c4 = pltpu.async_copy(dstp_hbm.at[cidxb], dstb, sem4)
                c1.wait()
                c2.wait()
                c5 = pltpu.async_copy(xl_hbm.at[sidxb], xlg, sem1)
                c6 = pltpu.async_copy(xr_hbm.at[didxb], xrg, sem2)
                c3.wait()
                c4.wait()
                c5.wait()
                c6.wait()

                @pl.loop(0, CGB, unroll=2)
                def _(e):
                    xs = []
                    lacc = None
                    for r in range(nreg):
                        s = pl.ds(r * LANES, LANES)
                        xv = xlg[e, s]
                        xs.append(xv)
                        z = xv + xrg[e, s] + eeg[e, s]
                        z = jnp.maximum(z, 0.2 * z)
                        t = z * atts[r]
                        lacc = t if lacc is None else lacc + t
                    logit = jnp.sum(lacc)
                    exv = jnp.exp(jnp.broadcast_to(logit, (LANES,)))
                    for r in range(nreg):
                        s = pl.ds(r * LANES, LANES)
                        xlg[e, s] = xs[r] * exv
                    exb[e, :] = exv

                for v in range(CGB // LANES):
                    inb = (iota + (base + v * LANES)) < pos
                    dstv = dstb[pl.ds(v * LANES, LANES)]
                    ridx = jnp.clip(dstv - lo, 0, rpt - 1)
                    erows = iota + v * LANES
                    exv16 = plsc.load_gather(exb, [erows, zero16i])
                    plsc.addupdate_scatter(den, [ridx, zero16i], exv16, mask=inb)

                    @pl.loop(0, 128, unroll=16)
                    def _(ch):
                        chs = jnp.broadcast_to(ch, (LANES,))
                        vals = plsc.load_gather(xlg, [erows, chs])
                        plsc.addupdate_scatter(acc, [ridx, chs], vals, mask=inb)

        pltpu.sync_copy(acc, uout.at[h, pl.ds(lo, rpt)])
        pltpu.sync_copy(den, exout.at[h, pl.ds(lo, rpt)])


def _gat_pass(dstp, gsrc, gdst, xl, xr, ee, att, N1, EP_pad):
    H, C = att.shape
    rpt = N1 // TILES
    mesh = plsc.VectorSubcoreMesh(core_axis_name="c", subcore_axis_name="s",
                                  num_cores=NC, num_subcores=NS)
    body = functools.partial(_gat_body, H=H, C=C, ep_pad=EP_pad, rpt=rpt)
    fn = pl.kernel(
        body,
        out_type=[
            jax.ShapeDtypeStruct((H, N1, C), jnp.float32),
            jax.ShapeDtypeStruct((H, N1, 16), jnp.float32),
        ],
        mesh=mesh,
        compiler_params=pltpu.CompilerParams(needs_layout_passes=False),
        scratch_types=[
            pltpu.VMEM((CAP,), jnp.int32),
            pltpu.VMEM((BK,), jnp.int32),
            pltpu.VMEM((CGB,), jnp.int32),
            pltpu.VMEM((CGB,), jnp.int32),
            pltpu.VMEM((CGB,), jnp.int32),
            pltpu.VMEM((CGB,), jnp.int32),
            pltpu.VMEM((CGB,), jnp.int32),
            pltpu.VMEM((CGB, C), jnp.float32),
            pltpu.VMEM((CGB, C), jnp.float32),
            pltpu.VMEM((CGB, C), jnp.float32),
            pltpu.VMEM((CGB, 16), jnp.float32),
            pltpu.VMEM((4, C), jnp.float32),
            pltpu.VMEM((N1 // TILES, C), jnp.float32),
            pltpu.VMEM((N1 // TILES, 16), jnp.float32),
            pltpu.SemaphoreType.DMA,
            pltpu.SemaphoreType.DMA,
            pltpu.SemaphoreType.DMA,
            pltpu.SemaphoreType.DMA,
        ],
    )
    return fn(dstp, gsrc, gdst, xl, xr, ee, att)


def _final_body(u_ref, e_ref, b_ref, out_ref):
    h = pl.program_id(1)
    bh = b_ref[pl.ds(h, 1), :]
    den = e_ref[0, :, 0:1]
    out_ref[...] = u_ref[0] / den + bh


def _finalize(uout, exout, bias2, N, H, C):
    T = 512
    n_tiles = _cdiv(N, T)
    grid = (n_tiles, H)
    out = pl.pallas_call(
        _final_body,
        grid=grid,
        in_specs=[
            pl.BlockSpec((1, T, C), lambda i, h: (h, i, 0)),
            pl.BlockSpec((1, T, 16), lambda i, h: (h, i, 0)),
            pl.BlockSpec(bias2.shape, lambda i, h: (0, 0)),
        ],
        out_specs=pl.BlockSpec((T, C), lambda i, h: (i, h)),
        out_shape=jax.ShapeDtypeStruct((n_tiles * T, H * C), jnp.float32),
    )(uout, exout, bias2)
    return out[:N]


def kernel(x, edge_index, edge_attr, Wl, bl, Wr, br, We, att, bias):
    N, D = x.shape
    E = edge_index.shape[1]
    H, C = att.shape
    EP = E + N
    EP_pad = _cdiv(EP, BK) * BK
    N1 = _cdiv(N + 1, 1024) * 1024
    NL = _cdiv(EP_pad - E, 512) * 512

    src = edge_index[0]
    dst = edge_index[1]
    loop_idx = jnp.arange(N, dtype=jnp.int32)
    pad_e = EP_pad - EP
    src_a = jnp.concatenate([src, loop_idx,
                             jnp.full((pad_e,), N, jnp.int32)])
    dst_a = jnp.concatenate([dst, loop_idx,
                             jnp.full((pad_e,), N, jnp.int32)])
    offs = (jnp.arange(H, dtype=jnp.int32) * N1)[:, None]
    gsrc = (src_a[None, :] + offs).reshape(-1)
    gdst = (dst_a[None, :] + offs).reshape(-1)

    x_pad = jnp.pad(x, ((0, N1 - N), (0, 0)))
    bl2 = bl.reshape(H, C)
    br2 = br.reshape(H, C)
    xl, xr = _node_proj(x_pad, Wl, bl2, Wr, br2, H, C, N1)
    xl = xl.reshape(H * N1, C)
    xr = xr.reshape(H * N1, C)

    ea_sum, deg = _ea_stats(dst_a, edge_attr, N1, EP_pad)
    ea_sum_p = jnp.pad(ea_sum, ((0, NL - N1), (0, 0)))
    deg_p = jnp.pad(deg, ((0, NL - N1), (0, 0)))
    ee_e = _ee_edges(edge_attr, We, H, C)
    ee_l = _ee_loops(ea_sum_p, deg_p, We, H, C)
    ee_l = ee_l[:, :EP_pad - E, :]
    ee = jnp.concatenate([ee_e, ee_l], axis=1).reshape(H * EP_pad, C)

    uout, exout = _gat_pass(dst_a, gsrc, gdst, xl, xr, ee, att, N1, EP_pad)

    out = _finalize(uout, exout, bias.reshape(H, C), N, H, C)
    return out


# speedup vs baseline: 1.0796x; 1.0796x over previous
import functools

import jax
import jax.numpy as jnp
from jax import lax
from jax.experimental import pallas as pl
from jax.experimental.pallas import tpu as pltpu
from jax.experimental.pallas import tpu_sc as plsc

NC = 2
NS = 16
LANES = 16
TILES = NC * NS
BK = 512
CGB = 64
CAP = 12288


def _cdiv(a, b):
    return (a + b - 1) // b


def _proj_body(x_ref, wl_ref, bl_ref, wr_ref, br_ref, xl_ref, xr_ref):
    xb = x_ref[...]
    h = pl.program_id(0)
    blh = bl_ref[pl.ds(h, 1), :]
    brh = br_ref[pl.ds(h, 1), :]
    xl_ref[0] = jnp.dot(xb, wl_ref[...],
                        preferred_element_type=jnp.float32) + blh
    xr_ref[0] = jnp.dot(xb, wr_ref[...],
                        preferred_element_type=jnp.float32) + brh


def _node_proj(x_pad, Wl, bl2, Wr, br2, H, C, N1):
    D = x_pad.shape[1]
    T = 512
    bl2_shape = bl2.shape
    grid = (H, N1 // T)
    return pl.pallas_call(
        _proj_body,
        grid=grid,
        in_specs=[
            pl.BlockSpec((T, D), lambda h, i: (i, 0)),
            pl.BlockSpec((D, C), lambda h, i: (0, h)),
            pl.BlockSpec(bl2_shape, lambda h, i: (0, 0)),
            pl.BlockSpec((D, C), lambda h, i: (0, h)),
            pl.BlockSpec(bl2_shape, lambda h, i: (0, 0)),
        ],
        out_specs=[
            pl.BlockSpec((1, T, C), lambda h, i: (h, i, 0)),
            pl.BlockSpec((1, T, C), lambda h, i: (h, i, 0)),
        ],
        out_shape=[
            jax.ShapeDtypeStruct((H, N1, C), jnp.float32),
            jax.ShapeDtypeStruct((H, N1, C), jnp.float32),
        ],
    )(x_pad, Wl, bl2, Wr, br2)


def _bucket(dstp_hbm, dbuf, listb, lo, hi, ep_pad):
    iota = lax.iota(jnp.int32, LANES)
    zero16i = jnp.zeros((LANES,), jnp.int32)

    @pl.loop(0, CAP // LANES)
    def _(k):
        listb[pl.ds(k * LANES, LANES)] = zero16i

    @pl.loop(0, ep_pad // BK, init_carry=jnp.int32(0))
    def pos(k, pos):
        pltpu.sync_copy(dstp_hbm.at[pl.ds(k * BK, BK)], dbuf)
        for v in range(BK // LANES):
            dv = dbuf[pl.ds(v * LANES, LANES)]
            ids = iota + (k * BK + v * LANES)
            m = (dv >= lo) & (dv < hi)
            ps = jnp.minimum(pos, CAP - LANES)
            plsc.store_compressed(listb.at[pl.ds(ps, LANES)], ids, mask=m)
            pc = plsc.all_reduce_population_count(m)
            pos = pos + pc[0]
        return pos

    return pos


def _ea_stats_body(dstp_hbm, ea_hbm, ea_out, deg_out,
                   listb, dbuf, cidxb, eabuf, dstb, accea, accdeg,
                   sem1, sem2, E, ep_pad, rpt):
    cid = lax.axis_index("c")
    sid = lax.axis_index("s")
    wid = sid * NC + cid
    lo = wid * rpt
    iota = lax.iota(jnp.int32, LANES)
    zero16f = jnp.zeros((LANES,), jnp.float32)
    zero16i = jnp.zeros((LANES,), jnp.int32)
    ones16f = jnp.full((LANES,), 1.0, jnp.float32)

    @pl.loop(0, rpt)
    def _(r):
        for c in range(8):
            accea[r, pl.ds(c * LANES, LANES)] = zero16f
        accdeg[r, :] = zero16f

    pos = _bucket(dstp_hbm, dbuf, listb, lo, lo + rpt, ep_pad)

    @pl.loop(0, CAP // CGB)
    def _(c):
        base = c * CGB

        @pl.when(base < pos)
        def _():
            for v in range(CGB // LANES):
                idv = listb[pl.ds(base + v * LANES, LANES)]
                cidxb[pl.ds(v * LANES, LANES)] = jnp.minimum(idv, E - 1)
            c1 = pltpu.async_copy(ea_hbm.at[cidxb], eabuf, sem1)
            c2 = pltpu.async_copy(dstp_hbm.at[cidxb], dstb, sem2)
            c1.wait()
            c2.wait()
            for v in range(CGB // LANES):
                idv = listb[pl.ds(base + v * LANES, LANES)]
                inb = (iota + (base + v * LANES)) < pos
                mk = inb & (idv < E)
                dstv = dstb[pl.ds(v * LANES, LANES)]
                ridx = jnp.clip(dstv - lo, 0, rpt - 1)
                plsc.addupdate_scatter(accdeg, [ridx, zero16i], ones16f, mask=mk)
                erows = iota + v * LANES

                @pl.loop(0, 128)
                def _(ch):
                    chs = jnp.broadcast_to(ch, (LANES,))
                    vals = plsc.load_gather(eabuf, [erows, chs])
                    plsc.addupdate_scatter(accea, [ridx, chs], vals, mask=mk)

    pltpu.sync_copy(accea, ea_out.at[pl.ds(lo, rpt)])
    pltpu.sync_copy(accdeg, deg_out.at[pl.ds(lo, rpt)])


def _ea_stats(dstp, edge_attr, N1, EP_pad):
    E, D = edge_attr.shape
    rpt = N1 // TILES
    mesh = plsc.VectorSubcoreMesh(core_axis_name="c", subcore_axis_name="s",
                                  num_cores=NC, num_subcores=NS)
    body = functools.partial(_ea_stats_body, E=E, ep_pad=EP_pad, rpt=rpt)
    fn = pl.kernel(
        body,
        out_type=[
            jax.ShapeDtypeStruct((N1, D), jnp.float32),
            jax.ShapeDtypeStruct((N1, 16), jnp.float32),
        ],
        mesh=mesh,
        compiler_params=pltpu.CompilerParams(needs_layout_passes=False),
        scratch_types=[
            pltpu.VMEM((CAP,), jnp.int32),
            pltpu.VMEM((BK,), jnp.int32),
            pltpu.VMEM((CGB,), jnp.int32),
            pltpu.VMEM((CGB, D), jnp.float32),
            pltpu.VMEM((CGB,), jnp.int32),
            pltpu.VMEM((N1 // TILES, D), jnp.float32),
            pltpu.VMEM((N1 // TILES, 16), jnp.float32),
            pltpu.SemaphoreType.DMA,
            pltpu.SemaphoreType.DMA,
        ],
    )
    return fn(dstp, edge_attr)


def _ee_edge_body(ea_ref, we_ref, out_ref):
    out_ref[0] = jnp.dot(ea_ref[...], we_ref[...],
                         preferred_element_type=jnp.float32)


def _ee_edges(edge_attr, We, H, C):
    E, D = edge_attr.shape
    T = 512
    grid = (H, E // T)
    return pl.pallas_call(
        _ee_edge_body,
        grid=grid,
        in_specs=[
            pl.BlockSpec((T, D), lambda h, i: (i, 0)),
            pl.BlockSpec((D, C), lambda h, i: (0, h)),
        ],
        out_specs=pl.BlockSpec((1, T, C), lambda h, i: (h, i, 0)),
        out_shape=jax.ShapeDtypeStruct((H, E, C), jnp.float32),
    )(edge_attr, We)


def _ee_loop_body(ea_ref, dg_ref, we_ref, out_ref):
    deg = jnp.maximum(dg_ref[:, 0:1], 1.0)
    mean = ea_ref[...] / deg
    out_ref[0] = jnp.dot(mean, we_ref[...],
                         preferred_element_type=jnp.float32)


def _ee_loops(ea_sum, deg, We, H, C):
    N1, D = ea_sum.shape
    T = 512
    grid = (H, N1 // T)
    return pl.pallas_call(
        _ee_loop_body,
        grid=grid,
        in_specs=[
            pl.BlockSpec((T, D), lambda h, i: (i, 0)),
            pl.BlockSpec((T, 16), lambda h, i: (i, 0)),
            pl.BlockSpec((D, C), lambda h, i: (0, h)),
        ],
        out_specs=pl.BlockSpec((1, T, C), lambda h, i: (h, i, 0)),
        out_shape=jax.ShapeDtypeStruct((H, N1, C), jnp.float32),
    )(ea_sum, deg, We)


def _gat_body(dstp_hbm, gsrc_hbm, gdst_hbm, xl_hbm, xr_hbm, ee_hbm, att_hbm,
              uout, exout,
              listb, dbuf, idxeb, cidxb, sidxb, didxb, dstb,
              xlg, xrg, eeg, exb, attbuf, acc, den,
              sem1, sem2, sem3, sem4,
              H, C, ep_pad, rpt):
    cid = lax.axis_index("c")
    sid = lax.axis_index("s")
    wid = sid * NC + cid
    lo = wid * rpt
    nreg = C // LANES
    iota = lax.iota(jnp.int32, LANES)
    zero16f = jnp.zeros((LANES,), jnp.float32)
    zero16i = jnp.zeros((LANES,), jnp.int32)

    pltpu.sync_copy(att_hbm, attbuf)
    pos = _bucket(dstp_hbm, dbuf, listb, lo, lo + rpt, ep_pad)

    for h in range(H):
        atts = [attbuf[h, pl.ds(r * LANES, LANES)] for r in range(nreg)]

        @pl.loop(0, rpt)
        def _(r):
            for c in range(nreg):
                acc[r, pl.ds(c * LANES, LANES)] = zero16f
            den[r, :] = zero16f

        @pl.loop(0, CAP // CGB)
        def _(c):
            base = c * CGB

            @pl.when(base < pos)
            def _():
                for v in range(CGB // LANES):
                    idv = listb[pl.ds(base + v * LANES, LANES)]
                    idxeb[pl.ds(v * LANES, LANES)] = idv + h * ep_pad
                    cidxb[pl.ds(v * LANES, LANES)] = idv
                c1 = pltpu.async_copy(gsrc_hbm.at[idxeb], sidxb, sem1)
                c2 = pltpu.async_copy(gdst_hbm.at[idxeb], didxb, sem2)
                c3 = pltpu.async_copy(ee_hbm.at[idxeb], eeg, sem3)
                c4 = pltpu.async_copy(dstp_hbm.at[cidxb], dstb, sem4)
                c1.wait()
                c2.wait()
                c5 = pltpu.async_copy(xl_hbm.at[sidxb], xlg, sem1)
                c6 = pltpu.async_copy(xr_hbm.at[didxb], xrg, sem2)
                c3.wait()
                c4.wait()
                c5.wait()
                c6.wait()

                @pl.loop(0, CGB)
                def _(e):
                    xs = []
                    lacc = None
                    for r in range(nreg):
                        s = pl.ds(r * LANES, LANES)
                        xv = xlg[e, s]
                        xs.append(xv)
                        z = xv + xrg[e, s] + eeg[e, s]
                        z = jnp.maximum(z, 0.2 * z)
                        t = z * atts[r]
                        lacc = t if lacc is None else lacc + t
                    logit = jnp.sum(lacc)
                    exv = jnp.exp(jnp.broadcast_to(logit, (LANES,)))
                    for r in range(nreg):
                        s = pl.ds(r * LANES, LANES)
                        xlg[e, s] = xs[r] * exv
                    exb[e, :] = exv

                for v in range(CGB // LANES):
                    inb = (iota + (base + v * LANES)) < pos
                    dstv = dstb[pl.ds(v * LANES, LANES)]
                    ridx = jnp.clip(dstv - lo, 0, rpt - 1)
                    erows = iota + v * LANES
                    exv16 = plsc.load_gather(exb, [erows, zero16i])
                    plsc.addupdate_scatter(den, [ridx, zero16i], exv16, mask=inb)

                    @pl.loop(0, 128)
                    def _(ch):
                        chs = jnp.broadcast_to(ch, (LANES,))
                        vals = plsc.load_gather(xlg, [erows, chs])
                        plsc.addupdate_scatter(acc, [ridx, chs], vals, mask=inb)

        pltpu.sync_copy(acc, uout.at[h, pl.ds(lo, rpt)])
        pltpu.sync_copy(den, exout.at[h, pl.ds(lo, rpt)])


def _gat_pass(dstp, gsrc, gdst, xl, xr, ee, att, N1, EP_pad):
    H, C = att.shape
    rpt = N1 // TILES
    mesh = plsc.VectorSubcoreMesh(core_axis_name="c", subcore_axis_name="s",
                                  num_cores=NC, num_subcores=NS)
    body = functools.partial(_gat_body, H=H, C=C, ep_pad=EP_pad, rpt=rpt)
    fn = pl.kernel(
        body,
        out_type=[
            jax.ShapeDtypeStruct((H, N1, C), jnp.float32),
            jax.ShapeDtypeStruct((H, N1, 16), jnp.float32),
        ],
        mesh=mesh,
        compiler_params=pltpu.CompilerParams(needs_layout_passes=False),
        scratch_types=[
            pltpu.VMEM((CAP,), jnp.int32),
            pltpu.VMEM((BK,), jnp.int32),
            pltpu.VMEM((CGB,), jnp.int32),
            pltpu.VMEM((CGB,), jnp.int32),
            pltpu.VMEM((CGB,), jnp.int32),
            pltpu.VMEM((CGB,), jnp.int32),
            pltpu.VMEM((CGB,), jnp.int32),
            pltpu.VMEM((CGB, C), jnp.float32),
            pltpu.VMEM((CGB, C), jnp.float32),
            pltpu.VMEM((CGB, C), jnp.float32),
            pltpu.VMEM((CGB, 16), jnp.float32),
            pltpu.VMEM((4, C), jnp.float32),
            pltpu.VMEM((N1 // TILES, C), jnp.float32),
            pltpu.VMEM((N1 // TILES, 16), jnp.float32),
            pltpu.SemaphoreType.DMA,
            pltpu.SemaphoreType.DMA,
            pltpu.SemaphoreType.DMA,
            pltpu.SemaphoreType.DMA,
        ],
    )
    return fn(dstp, gsrc, gdst, xl, xr, ee, att)


def _final_body(u_ref, e_ref, b_ref, out_ref):
    h = pl.program_id(1)
    bh = b_ref[pl.ds(h, 1), :]
    den = e_ref[0, :, 0:1]
    out_ref[...] = u_ref[0] / den + bh


def _finalize(uout, exout, bias2, N, H, C):
    T = 512
    n_tiles = _cdiv(N, T)
    grid = (n_tiles, H)
    out = pl.pallas_call(
        _final_body,
        grid=grid,
        in_specs=[
            pl.BlockSpec((1, T, C), lambda i, h: (h, i, 0)),
            pl.BlockSpec((1, T, 16), lambda i, h: (h, i, 0)),
            pl.BlockSpec(bias2.shape, lambda i, h: (0, 0)),
        ],
        out_specs=pl.BlockSpec((T, C), lambda i, h: (i, h)),
        out_shape=jax.ShapeDtypeStruct((n_tiles * T, H * C), jnp.float32),
    )(uout, exout, bias2)
    return out[:N]


def kernel(x, edge_index, edge_attr, Wl, bl, Wr, br, We, att, bias):
    N, D = x.shape
    E = edge_index.shape[1]
    H, C = att.shape
    EP = E + N
    EP_pad = _cdiv(EP, BK) * BK
    N1 = _cdiv(N + 1, 1024) * 1024
    NL = _cdiv(EP_pad - E, 512) * 512

    src = edge_index[0]
    dst = edge_index[1]
    loop_idx = jnp.arange(N, dtype=jnp.int32)
    pad_e = EP_pad - EP
    src_a = jnp.concatenate([src, loop_idx,
                             jnp.full((pad_e,), N, jnp.int32)])
    dst_a = jnp.concatenate([dst, loop_idx,
                             jnp.full((pad_e,), N, jnp.int32)])
    offs = (jnp.arange(H, dtype=jnp.int32) * N1)[:, None]
    gsrc = (src_a[None, :] + offs).reshape(-1)
    gdst = (dst_a[None, :] + offs).reshape(-1)

    x_pad = jnp.pad(x, ((0, N1 - N), (0, 0)))
    bl2 = bl.reshape(H, C)
    br2 = br.reshape(H, C)
    xl, xr = _node_proj(x_pad, Wl, bl2, Wr, br2, H, C, N1)
    xl = xl.reshape(H * N1, C)
    xr = xr.reshape(H * N1, C)

    ea_sum, deg = _ea_stats(dst_a, edge_attr, N1, EP_pad)
    ea_sum_p = jnp.pad(ea_sum, ((0, NL - N1), (0, 0)))
    deg_p = jnp.pad(deg, ((0, NL - N1), (0, 0)))
    ee_e = _ee_edges(edge_attr, We, H, C)
    ee_l = _ee_loops(ea_sum_p, deg_p, We, H, C)
    ee_l = ee_l[:, :EP_pad - E, :]
    ee = jnp.concatenate([ee_e, ee_l], axis=1).reshape(H * EP_pad, C)

    uout, exout = _gat_pass(dst_a, gsrc, gdst, xl, xr, ee, att, N1, EP_pad)

    out = _finalize(uout, exout, bias.reshape(H, C), N, H, C)
    return out


# speedup vs baseline: 1.0868x; 1.0068x over previous
import functools

import jax
import jax.numpy as jnp
from jax import lax
from jax.experimental import pallas as pl
from jax.experimental.pallas import tpu as pltpu
from jax.experimental.pallas import tpu_sc as plsc

NC = 2
NS = 16
LANES = 16
TILES = NC * NS
BK = 512
CGB = 32
CAP = 12288


def _cdiv(a, b):
    return (a + b - 1) // b


def _proj_body(x_ref, wl_ref, bl_ref, wr_ref, br_ref, xl_ref, xr_ref):
    xb = x_ref[...]
    h = pl.program_id(0)
    blh = bl_ref[pl.ds(h, 1), :]
    brh = br_ref[pl.ds(h, 1), :]
    xl_ref[0] = jnp.dot(xb, wl_ref[...],
                        preferred_element_type=jnp.float32) + blh
    xr_ref[0] = jnp.dot(xb, wr_ref[...],
                        preferred_element_type=jnp.float32) + brh


def _node_proj(x_pad, Wl, bl2, Wr, br2, H, C, N1):
    D = x_pad.shape[1]
    T = 512
    bl2_shape = bl2.shape
    grid = (H, N1 // T)
    return pl.pallas_call(
        _proj_body,
        grid=grid,
        in_specs=[
            pl.BlockSpec((T, D), lambda h, i: (i, 0)),
            pl.BlockSpec((D, C), lambda h, i: (0, h)),
            pl.BlockSpec(bl2_shape, lambda h, i: (0, 0)),
            pl.BlockSpec((D, C), lambda h, i: (0, h)),
            pl.BlockSpec(bl2_shape, lambda h, i: (0, 0)),
        ],
        out_specs=[
            pl.BlockSpec((1, T, C), lambda h, i: (h, i, 0)),
            pl.BlockSpec((1, T, C), lambda h, i: (h, i, 0)),
        ],
        out_shape=[
            jax.ShapeDtypeStruct((H, N1, C), jnp.float32),
            jax.ShapeDtypeStruct((H, N1, C), jnp.float32),
        ],
    )(x_pad, Wl, bl2, Wr, br2)


def _bucket(dstp_hbm, dbuf, listb, lo, hi, ep_pad):
    iota = lax.iota(jnp.int32, LANES)
    zero16i = jnp.zeros((LANES,), jnp.int32)

    @pl.loop(0, CAP // LANES)
    def _(k):
        listb[pl.ds(k * LANES, LANES)] = zero16i

    @pl.loop(0, ep_pad // BK, init_carry=jnp.int32(0))
    def pos(k, pos):
        pltpu.sync_copy(dstp_hbm.at[pl.ds(k * BK, BK)], dbuf)
        for v in range(BK // LANES):
            dv = dbuf[pl.ds(v * LANES, LANES)]
            ids = iota + (k * BK + v * LANES)
            m = (dv >= lo) & (dv < hi)
            ps = jnp.minimum(pos, CAP - LANES)
            plsc.store_compressed(listb.at[pl.ds(ps, LANES)], ids, mask=m)
            pc = plsc.all_reduce_population_count(m)
            pos = pos + pc[0]
        return pos

    return pos


def _ea_stats_body(dstp_hbm, ea_hbm, ea_out, deg_out,
                   listb, dbuf, cidxb, eabuf, dstb, accea, accdeg,
                   sem1, sem2, E, ep_pad, rpt):
    cid = lax.axis_index("c")
    sid = lax.axis_index("s")
    wid = sid * NC + cid
    lo = wid * rpt
    iota = lax.iota(jnp.int32, LANES)
    zero16f = jnp.zeros((LANES,), jnp.float32)
    zero16i = jnp.zeros((LANES,), jnp.int32)
    ones16f = jnp.full((LANES,), 1.0, jnp.float32)

    @pl.loop(0, rpt)
    def _(r):
        for c in range(8):
            accea[r, pl.ds(c * LANES, LANES)] = zero16f
        accdeg[r, :] = zero16f

    pos = _bucket(dstp_hbm, dbuf, listb, lo, lo + rpt, ep_pad)

    @pl.loop(0, CAP // CGB)
    def _(c):
        base = c * CGB

        @pl.when(base < pos)
        def _():
            for v in range(CGB // LANES):
                idv = listb[pl.ds(base + v * LANES, LANES)]
                cidxb[pl.ds(v * LANES, LANES)] = jnp.minimum(idv, E - 1)
            c1 = pltpu.async_copy(ea_hbm.at[cidxb], eabuf, sem1)
            c2 = pltpu.async_copy(dstp_hbm.at[cidxb], dstb, sem2)
            c1.wait()
            c2.wait()
            for v in range(CGB // LANES):
                idv = listb[pl.ds(base + v * LANES, LANES)]
                inb = (iota + (base + v * LANES)) < pos
                mk = inb & (idv < E)
                dstv = dstb[pl.ds(v * LANES, LANES)]
                ridx = jnp.clip(dstv - lo, 0, rpt - 1)
                plsc.addupdate_scatter(accdeg, [ridx, zero16i], ones16f, mask=mk)
                erows = iota + v * LANES

                @pl.loop(0, 128)
                def _(ch):
                    chs = jnp.broadcast_to(ch, (LANES,))
                    vals = plsc.load_gather(eabuf, [erows, chs])
                    plsc.addupdate_scatter(accea, [ridx, chs], vals, mask=mk)

    pltpu.sync_copy(accea, ea_out.at[pl.ds(lo, rpt)])
    pltpu.sync_copy(accdeg, deg_out.at[pl.ds(lo, rpt)])


def _ea_stats(dstp, edge_attr, N1, EP_pad):
    E, D = edge_attr.shape
    rpt = N1 // TILES
    mesh = plsc.VectorSubcoreMesh(core_axis_name="c", subcore_axis_name="s",
                                  num_cores=NC, num_subcores=NS)
    body = functools.partial(_ea_stats_body, E=E, ep_pad=EP_pad, rpt=rpt)
    fn = pl.kernel(
        body,
        out_type=[
            jax.ShapeDtypeStruct((N1, D), jnp.float32),
            jax.ShapeDtypeStruct((N1, 16), jnp.float32),
        ],
        mesh=mesh,
        compiler_params=pltpu.CompilerParams(needs_layout_passes=False),
        scratch_types=[
            pltpu.VMEM((CAP,), jnp.int32),
            pltpu.VMEM((BK,), jnp.int32),
            pltpu.VMEM((CGB,), jnp.int32),
            pltpu.VMEM((CGB, D), jnp.float32),
            pltpu.VMEM((CGB,), jnp.int32),
            pltpu.VMEM((N1 // TILES, D), jnp.float32),
            pltpu.VMEM((N1 // TILES, 16), jnp.float32),
            pltpu.SemaphoreType.DMA,
            pltpu.SemaphoreType.DMA,
        ],
    )
    return fn(dstp, edge_attr)


def _ee_edge_body(ea_ref, we_ref, out_ref):
    out_ref[0] = jnp.dot(ea_ref[...], we_ref[...],
                         preferred_element_type=jnp.float32)


def _ee_edges(edge_attr, We, H, C):
    E, D = edge_attr.shape
    T = 512
    grid = (H, E // T)
    return pl.pallas_call(
        _ee_edge_body,
        grid=grid,
        in_specs=[
            pl.BlockSpec((T, D), lambda h, i: (i, 0)),
            pl.BlockSpec((D, C), lambda h, i: (0, h)),
        ],
        out_specs=pl.BlockSpec((1, T, C), lambda h, i: (h, i, 0)),
        out_shape=jax.ShapeDtypeStruct((H, E, C), jnp.float32),
    )(edge_attr, We)


def _ee_loop_body(ea_ref, dg_ref, we_ref, out_ref):
    deg = jnp.maximum(dg_ref[:, 0:1], 1.0)
    mean = ea_ref[...] / deg
    out_ref[0] = jnp.dot(mean, we_ref[...],
                         preferred_element_type=jnp.float32)


def _ee_loops(ea_sum, deg, We, H, C):
    N1, D = ea_sum.shape
    T = 512
    grid = (H, N1 // T)
    return pl.pallas_call(
        _ee_loop_body,
        grid=grid,
        in_specs=[
            pl.BlockSpec((T, D), lambda h, i: (i, 0)),
            pl.BlockSpec((T, 16), lambda h, i: (i, 0)),
            pl.BlockSpec((D, C), lambda h, i: (0, h)),
        ],
        out_specs=pl.BlockSpec((1, T, C), lambda h, i: (h, i, 0)),
        out_shape=jax.ShapeDtypeStruct((H, N1, C), jnp.float32),
    )(ea_sum, deg, We)


def _gat_body(dstp_hbm, gsrc_hbm, gdst_hbm, xl_hbm, xr_hbm, ee_hbm, att_hbm,
              uout, exout,
              listb, dbuf, idxeb, cidxb, sidxb, didxb, dstb,
              xlg, xrg, eeg, exb,
              idxeb2, cidxb2, sidxb2, didxb2, dstb2,
              xlg2, xrg2, eeg2, exb2, attbuf, acc, den,
              sem1, sem2, sem3, sem4, sem5, sem6, sem7, sem8,
              H, C, ep_pad, rpt):
    cid = lax.axis_index("c")
    sid = lax.axis_index("s")
    wid = sid * NC + cid
    lo = wid * rpt
    nreg = C // LANES
    iota = lax.iota(jnp.int32, LANES)
    zero16f = jnp.zeros((LANES,), jnp.float32)
    zero16i = jnp.zeros((LANES,), jnp.int32)

    pltpu.sync_copy(att_hbm, attbuf)
    pos = _bucket(dstp_hbm, dbuf, listb, lo, lo + rpt, ep_pad)

    for h in range(H):
        atts = [attbuf[h, pl.ds(r * LANES, LANES)] for r in range(nreg)]

        @pl.loop(0, rpt)
        def _(r):
            for c in range(nreg):
                acc[r, pl.ds(c * LANES, LANES)] = zero16f
            den[r, :] = zero16f

        bufs = ((idxeb, cidxb, sidxb, didxb, dstb, xlg, xrg, eeg, exb,
                 sem1, sem2, sem3, sem4),
                (idxeb2, cidxb2, sidxb2, didxb2, dstb2, xlg2, xrg2, eeg2, exb2,
                 sem5, sem6, sem7, sem8))

        def stage1(base, b):
            (bidxe, bcidx, bsidx, bdidx, bdst, bxl, bxr, bee, bex,
             s1, s2, s3, s4) = bufs[b]

            @pl.when(base < pos)
            def _():
                for v in range(CGB // LANES):
                    idv = listb[pl.ds(base + v * LANES, LANES)]
                    bidxe[pl.ds(v * LANES, LANES)] = idv + h * ep_pad
                    bcidx[pl.ds(v * LANES, LANES)] = idv
                pltpu.async_copy(gsrc_hbm.at[bidxe], bsidx, s1)
                pltpu.async_copy(gdst_hbm.at[bidxe], bdidx, s2)
                pltpu.async_copy(ee_hbm.at[bidxe], bee, s3)
                pltpu.async_copy(dstp_hbm.at[bcidx], bdst, s4)

        def stage2(base, b):
            (bidxe, bcidx, bsidx, bdidx, bdst, bxl, bxr, bee, bex,
             s1, s2, s3, s4) = bufs[b]

            @pl.when(base < pos)
            def _():
                pltpu.make_async_copy(gsrc_hbm.at[bidxe], bsidx, s1).wait()
                pltpu.make_async_copy(gdst_hbm.at[bidxe], bdidx, s2).wait()
                pltpu.async_copy(xl_hbm.at[bsidx], bxl, s1)
                pltpu.async_copy(xr_hbm.at[bdidx], bxr, s2)

        def stage3(base, b):
            (bidxe, bcidx, bsidx, bdidx, bdst, bxl, bxr, bee, bex,
             s1, s2, s3, s4) = bufs[b]

            @pl.when(base < pos)
            def _():
                pltpu.make_async_copy(ee_hbm.at[bidxe], bee, s3).wait()
                pltpu.make_async_copy(dstp_hbm.at[bcidx], bdst, s4).wait()
                pltpu.make_async_copy(xl_hbm.at[bsidx], bxl, s1).wait()
                pltpu.make_async_copy(xr_hbm.at[bdidx], bxr, s2).wait()

                @pl.loop(0, CGB)
                def _(e):
                    xs = []
                    lacc = None
                    for r in range(nreg):
                        s = pl.ds(r * LANES, LANES)
                        xv = bxl[e, s]
                        xs.append(xv)
                        z = xv + bxr[e, s] + bee[e, s]
                        z = jnp.maximum(z, 0.2 * z)
                        t = z * atts[r]
                        lacc = t if lacc is None else lacc + t
                    logit = jnp.sum(lacc)
                    exv = jnp.exp(jnp.broadcast_to(logit, (LANES,)))
                    for r in range(nreg):
                        s = pl.ds(r * LANES, LANES)
                        bxl[e, s] = xs[r] * exv
                    bex[e, :] = exv

                for v in range(CGB // LANES):
                    inb = (iota + (base + v * LANES)) < pos
                    dstv = bdst[pl.ds(v * LANES, LANES)]
                    ridx = jnp.clip(dstv - lo, 0, rpt - 1)
                    erows = iota + v * LANES
                    exv16 = plsc.load_gather(bex, [erows, zero16i])
                    plsc.addupdate_scatter(den, [ridx, zero16i], exv16, mask=inb)

                    @pl.loop(0, 128)
                    def _(ch):
                        chs = jnp.broadcast_to(ch, (LANES,))
                        vals = plsc.load_gather(bxl, [erows, chs])
                        plsc.addupdate_scatter(acc, [ridx, chs], vals, mask=inb)

        @pl.loop(0, CAP // (2 * CGB))
        def _(c):
            b0 = (2 * c) * CGB
            b1 = (2 * c + 1) * CGB
            stage1(b0, 0)
            stage1(b1, 1)
            stage2(b0, 0)
            stage2(b1, 1)
            stage3(b0, 0)
            stage3(b1, 1)

        pltpu.sync_copy(acc, uout.at[h, pl.ds(lo, rpt)])
        pltpu.sync_copy(den, exout.at[h, pl.ds(lo, rpt)])


def _gat_pass(dstp, gsrc, gdst, xl, xr, ee, att, N1, EP_pad):
    H, C = att.shape
    rpt = N1 // TILES
    mesh = plsc.VectorSubcoreMesh(core_axis_name="c", subcore_axis_name="s",
                                  num_cores=NC, num_subcores=NS)
    body = functools.partial(_gat_body, H=H, C=C, ep_pad=EP_pad, rpt=rpt)
    fn = pl.kernel(
        body,
        out_type=[
            jax.ShapeDtypeStruct((H, N1, C), jnp.float32),
            jax.ShapeDtypeStruct((H, N1, 16), jnp.float32),
        ],
        mesh=mesh,
        compiler_params=pltpu.CompilerParams(needs_layout_passes=False),
        scratch_types=(
            [
                pltpu.VMEM((CAP,), jnp.int32),
                pltpu.VMEM((BK,), jnp.int32),
            ]
            + [pltpu.VMEM((CGB,), jnp.int32) for _ in range(5)]
            + [pltpu.VMEM((CGB, C), jnp.float32) for _ in range(3)]
            + [pltpu.VMEM((CGB, 16), jnp.float32)]
            + [pltpu.VMEM((CGB,), jnp.int32) for _ in range(5)]
            + [pltpu.VMEM((CGB, C), jnp.float32) for _ in range(3)]
            + [pltpu.VMEM((CGB, 16), jnp.float32)]
            + [
                pltpu.VMEM((4, C), jnp.float32),
                pltpu.VMEM((N1 // TILES, C), jnp.float32),
                pltpu.VMEM((N1 // TILES, 16), jnp.float32),
            ]
            + [pltpu.SemaphoreType.DMA for _ in range(8)]
        ),
    )
    return fn(dstp, gsrc, gdst, xl, xr, ee, att)


def _final_body(u_ref, e_ref, b_ref, out_ref):
    h = pl.program_id(1)
    bh = b_ref[pl.ds(h, 1), :]
    den = e_ref[0, :, 0:1]
    out_ref[...] = u_ref[0] / den + bh


def _finalize(uout, exout, bias2, N, H, C):
    T = 512
    n_tiles = _cdiv(N, T)
    grid = (n_tiles, H)
    out = pl.pallas_call(
        _final_body,
        grid=grid,
        in_specs=[
            pl.BlockSpec((1, T, C), lambda i, h: (h, i, 0)),
            pl.BlockSpec((1, T, 16), lambda i, h: (h, i, 0)),
            pl.BlockSpec(bias2.shape, lambda i, h: (0, 0)),
        ],
        out_specs=pl.BlockSpec((T, C), lambda i, h: (i, h)),
        out_shape=jax.ShapeDtypeStruct((n_tiles * T, H * C), jnp.float32),
    )(uout, exout, bias2)
    return out[:N]


def kernel(x, edge_index, edge_attr, Wl, bl, Wr, br, We, att, bias):
    N, D = x.shape
    E = edge_index.shape[1]
    H, C = att.shape
    EP = E + N
    EP_pad = _cdiv(EP, BK) * BK
    N1 = _cdiv(N + 1, 1024) * 1024
    NL = _cdiv(EP_pad - E, 512) * 512

    src = edge_index[0]
    dst = edge_index[1]
    loop_idx = jnp.arange(N, dtype=jnp.int32)
    pad_e = EP_pad - EP
    src_a = jnp.concatenate([src, loop_idx,
                             jnp.full((pad_e,), N, jnp.int32)])
    dst_a = jnp.concatenate([dst, loop_idx,
                             jnp.full((pad_e,), N, jnp.int32)])
    offs = (jnp.arange(H, dtype=jnp.int32) * N1)[:, None]
    gsrc = (src_a[None, :] + offs).reshape(-1)
    gdst = (dst_a[None, :] + offs).reshape(-1)

    x_pad = jnp.pad(x, ((0, N1 - N), (0, 0)))
    bl2 = bl.reshape(H, C)
    br2 = br.reshape(H, C)
    xl, xr = _node_proj(x_pad, Wl, bl2, Wr, br2, H, C, N1)
    xl = xl.reshape(H * N1, C)
    xr = xr.reshape(H * N1, C)

    ea_sum, deg = _ea_stats(dst_a, edge_attr, N1, EP_pad)
    ea_sum_p = jnp.pad(ea_sum, ((0, NL - N1), (0, 0)))
    deg_p = jnp.pad(deg, ((0, NL - N1), (0, 0)))
    ee_e = _ee_edges(edge_attr, We, H, C)
    ee_l = _ee_loops(ea_sum_p, deg_p, We, H, C)
    ee_l = ee_l[:, :EP_pad - E, :]
    ee = jnp.concatenate([ee_e, ee_l], axis=1).reshape(H * EP_pad, C)

    uout, exout = _gat_pass(dst_a, gsrc, gdst, xl, xr, ee, att, N1, EP_pad)

    out = _finalize(uout, exout, bias.reshape(H, C), N, H, C)
    return out
